# Initial kernel scaffold; baseline (speedup 1.0000x reference)
import jax
import jax.numpy as jnp
from jax.experimental import pallas as pl


def kernel(inputs, embedding_matrix):
    raise NotImplementedError("write your pallas kernel here")


# speedup vs baseline: 1.5054x; 1.5054x over previous
import functools

import jax
import jax.numpy as jnp
from jax import lax
from jax.experimental import pallas as pl
from jax.experimental.pallas import tpu as pltpu
from jax.experimental.pallas import tpu_sc as plsc

VOCAB = 1000000
EMBED_DIM = 32
BATCH = 4096
HIST = 200

NC = 2
NS = 16
NW = NC * NS

B = BATCH * HIST
GROUP = 128
N_PER_W = B // NW
GROUPS_PER_W = N_PER_W // GROUP
GROUPS_PER_CHUNK = 10
CHUNK = GROUPS_PER_CHUNK * GROUP
N_CHUNKS = GROUPS_PER_W // GROUPS_PER_CHUNK

_mesh = plsc.VectorSubcoreMesh(
    core_axis_name="c", subcore_axis_name="s", num_cores=NC, num_subcores=NS
)


@functools.partial(
    pl.kernel,
    out_type=jax.ShapeDtypeStruct((B, EMBED_DIM), jnp.float32),
    mesh=_mesh,
    scratch_types=[
        pltpu.VMEM((GROUPS_PER_W, GROUP), jnp.int32),
        pltpu.VMEM((2, CHUNK, EMBED_DIM), jnp.float32),
        pltpu.SemaphoreType.DMA,
        pltpu.SemaphoreType.DMA,
        pltpu.SemaphoreType.DMA,
        pltpu.SemaphoreType.DMA,
    ],
    compiler_params=pltpu.CompilerParams(use_tc_tiling_on_sc=False),
)
def _emb_lookup(idx_hbm, table_hbm, out_hbm, idx_v, rows_v, g0, g1, s0, s1):
    wid = lax.axis_index("s") * NC + lax.axis_index("c")
    base = wid * N_PER_W
    gsems = (g0, g1)
    ssems = (s0, s1)

    pltpu.sync_copy(idx_hbm.at[pl.ds(wid * GROUPS_PER_W, GROUPS_PER_W)], idx_v)

    def start_gathers(chunk, buf):
        for j in range(GROUPS_PER_CHUNK):
            pltpu.async_copy(
                table_hbm.at[idx_v.at[chunk * GROUPS_PER_CHUNK + j]],
                rows_v.at[buf, pl.ds(j * GROUP, GROUP)],
                gsems[buf],
            )

    def wait_gathers(chunk, buf):
        for j in range(GROUPS_PER_CHUNK):
            pltpu.make_async_copy(
                table_hbm.at[idx_v.at[chunk * GROUPS_PER_CHUNK + j]],
                rows_v.at[buf, pl.ds(j * GROUP, GROUP)],
                gsems[buf],
            ).wait()

    def store(chunk, buf):
        return pltpu.async_copy(
            rows_v.at[buf],
            out_hbm.at[pl.ds(base + chunk * CHUNK, CHUNK)],
            ssems[buf],
        )

    def wait_store(chunk, buf):
        pltpu.make_async_copy(
            rows_v.at[buf],
            out_hbm.at[pl.ds(base + chunk * CHUNK, CHUNK)],
            ssems[buf],
        ).wait()

    start_gathers(0, 0)

    @pl.loop(0, N_CHUNKS // 2)
    def _body(p):
        c = p * 2
        start_gathers(c + 1, 1)
        wait_gathers(c, 0)
        store(c, 0)
        @pl.when(p < N_CHUNKS // 2 - 1)
        def _():
            wait_store(c, 0)
            start_gathers(c + 2, 0)
        wait_gathers(c + 1, 1)
        store(c + 1, 1)
        @pl.when(p < N_CHUNKS // 2 - 1)
        def _():
            wait_store(c + 1, 1)

    wait_store(N_CHUNKS - 2, 0)
    wait_store(N_CHUNKS - 1, 1)


def kernel(inputs, embedding_matrix):
    idx = inputs.reshape(NW * GROUPS_PER_W, GROUP).astype(jnp.int32)
    out = _emb_lookup(idx, embedding_matrix)
    return out.reshape(BATCH, HIST, EMBED_DIM)
